# baseline (speedup 1.0000x reference)
import functools

import jax
import jax.numpy as jnp
from jax import lax
from jax.experimental import pallas as pl
from jax.experimental.pallas import tpu as pltpu
from jax.experimental.pallas import tpu_sc as plsc

_B, _P, _D = 64, 1024, 768
_TC_B = 36
_SC_B = _B - _TC_B
_NC, _NS, _L = 2, 16, 16
_NW = _NC * _NS
_PW = _P // _NW
_NCHUNK = _D // _L


def _compute(x_v, pos_v, o_v):
    def row_body(r, carry):
        for c in range(_NCHUNK):
            sl = pl.ds(c * _L, _L)
            o_v[r, sl] = x_v[r, sl] + pos_v[r, sl]
        return carry

    lax.fori_loop(0, _PW, row_body, 0)


def _sc_kernel_body(x_hbm, pos_hbm, out_hbm, pos_v, x0, x1, o0, o1,
                    in_sem0, in_sem1, out_sem0, out_sem1):
    wid = lax.axis_index("s") * _NC + lax.axis_index("c")
    ps = wid * _PW
    psl = pl.ds(ps, _PW)

    def in_slice(b):
        return x_hbm.at[_TC_B + b, psl]

    def out_slice(b):
        return out_hbm.at[b, psl]

    def step(b, x_v, o_v, in_sem, out_sem, first, last):
        pltpu.make_async_copy(in_slice(b), x_v, in_sem).wait()
        if not first:
            pltpu.make_async_copy(o_v, out_slice(b - 2), out_sem).wait()
        _compute(x_v, pos_v, o_v)
        pltpu.async_copy(o_v, out_slice(b), out_sem)
        if not last:
            pltpu.async_copy(in_slice(b + 2), x_v, in_sem)

    pltpu.sync_copy(pos_hbm.at[psl], pos_v)
    pltpu.async_copy(in_slice(0), x0, in_sem0)
    pltpu.async_copy(in_slice(1), x1, in_sem1)

    step(0, x0, o0, in_sem0, out_sem0, first=True, last=False)
    step(1, x1, o1, in_sem1, out_sem1, first=True, last=False)

    def pair_body(i, carry):
        b0 = 2 * i
        step(b0, x0, o0, in_sem0, out_sem0, first=False, last=False)
        step(b0 + 1, x1, o1, in_sem1, out_sem1, first=False, last=False)
        return carry

    lax.fori_loop(1, _SC_B // 2 - 1, pair_body, 0)

    step(_SC_B - 2, x0, o0, in_sem0, out_sem0, first=False, last=True)
    step(_SC_B - 1, x1, o1, in_sem1, out_sem1, first=False, last=True)

    pltpu.make_async_copy(o0, out_slice(_SC_B - 2), out_sem0).wait()
    pltpu.make_async_copy(o1, out_slice(_SC_B - 1), out_sem1).wait()


@functools.partial(
    pl.kernel,
    out_type=jax.ShapeDtypeStruct((_SC_B, _P, _D), jnp.float32),
    mesh=plsc.VectorSubcoreMesh(
        core_axis_name="c", subcore_axis_name="s",
        num_cores=_NC, num_subcores=_NS,
    ),
    scratch_types=[
        pltpu.VMEM((_PW, _D), jnp.float32),
        pltpu.VMEM((_PW, _D), jnp.float32),
        pltpu.VMEM((_PW, _D), jnp.float32),
        pltpu.VMEM((_PW, _D), jnp.float32),
        pltpu.VMEM((_PW, _D), jnp.float32),
        pltpu.SemaphoreType.DMA,
        pltpu.SemaphoreType.DMA,
        pltpu.SemaphoreType.DMA,
        pltpu.SemaphoreType.DMA,
    ],
)
def _sc_kernel(x_hbm, pos_hbm, out_hbm, pos_v, x0, x1, o0, o1,
               in_sem0, in_sem1, out_sem0, out_sem1):
    _sc_kernel_body(x_hbm, pos_hbm, out_hbm, pos_v, x0, x1, o0, o1,
                    in_sem0, in_sem1, out_sem0, out_sem1)


def _tc_body(x_ref, p_ref, o_ref):
    o_ref[...] = x_ref[...] + p_ref[...]


def _tc_part(x2, pos_table):
    return pl.pallas_call(
        _tc_body,
        grid=(_TC_B,),
        in_specs=[
            pl.BlockSpec((_P, _D), lambda i: (i, 0)),
            pl.BlockSpec((_P, _D), lambda i: (0, 0)),
        ],
        out_specs=pl.BlockSpec((_P, _D), lambda i: (i, 0)),
        out_shape=jax.ShapeDtypeStruct((_TC_B * _P, _D), jnp.float32),
    )(x2, pos_table)


def kernel(encoded_patches, pos_table):
    x2 = encoded_patches.reshape(_B * _P, _D)
    out_tc = _tc_part(x2, pos_table).reshape(_TC_B, _P, _D)
    out_sc = _sc_kernel(encoded_patches, pos_table)
    return jnp.concatenate([out_tc, out_sc], axis=0)


# speedup vs baseline: 2.1021x; 2.1021x over previous
import jax
import jax.numpy as jnp
from jax.experimental import pallas as pl

_B, _P, _D = 64, 1024, 768


def _tc_body(x_ref, p_ref, o_ref):
    o_ref[...] = x_ref[...] + p_ref[...]


def kernel(encoded_patches, pos_table):
    x2 = encoded_patches.reshape(_B * _P, _D)
    out = pl.pallas_call(
        _tc_body,
        grid=(_B,),
        in_specs=[
            pl.BlockSpec((_P, _D), lambda i: (i, 0)),
            pl.BlockSpec((_P, _D), lambda i: (0, 0)),
        ],
        out_specs=pl.BlockSpec((_P, _D), lambda i: (i, 0)),
        out_shape=jax.ShapeDtypeStruct((_B * _P, _D), jnp.float32),
    )(x2, pos_table)
    return out.reshape(_B, _P, _D)
